# baseline (speedup 1.0000x reference)
import functools

import jax
import jax.numpy as jnp
from jax import lax
from jax.experimental import pallas as pl
from jax.experimental.pallas import tpu as pltpu
from jax.experimental.pallas import tpu_sc as plsc

_BATCH = 16384
_ROWS = 1000001
_MF = 16
_MLP = 32
_NC = 2
_NS = 16
_NW = _NC * _NS
_BPW = _BATCH // _NW
_CH = 128
_NCH = _BPW // _CH
_L = 16
_MVW = 16384
_NPR = 7936
_MVOUT = _NPR * 128


def _mv_body(t_ref, w_ref, out_ref):
    t = t_ref[...]
    w = w_ref[...]
    out_ref[...] = jnp.sum(t * w[:, None], axis=0).reshape(-1, 128)


def _matvec(table_t, w):
    return pl.pallas_call(
        _mv_body,
        grid=(pl.cdiv(_MVOUT, _MVW),),
        in_specs=[pl.BlockSpec((_MLP, _MVW), lambda j: (0, j)),
                  pl.BlockSpec((_MLP,), lambda j: (0,))],
        out_specs=pl.BlockSpec((_MVW // 128, 128), lambda j: (j, 0)),
        out_shape=jax.ShapeDtypeStruct((_NPR, 128), jnp.float32),
    )(table_t, w)


def _cp_body(in_ref, out_ref, sem):
    cp = pltpu.make_async_copy(in_ref, out_ref.at[pl.ds(0, _ROWS)], sem)
    cp.start()
    cp.wait()


def _hbm_pad8(table):
    d = table.shape[1]
    return pl.pallas_call(
        _cp_body,
        in_specs=[pl.BlockSpec(memory_space=pltpu.MemorySpace.HBM)],
        out_specs=pl.BlockSpec(memory_space=pltpu.MemorySpace.HBM),
        out_shape=jax.ShapeDtypeStruct((_ROWS + 7, d), jnp.float32),
        scratch_shapes=[pltpu.SemaphoreType.DMA],
    )(table)


def _sc_body(uix_hbm, mix_hbm, umf_hbm, mmf_hbm, pu_hbm, pm_hbm, w_hbm,
             out_hbm,
             idxu_v, idxm_v, gpu_v, gpm_v, umf_v, mmf_v, pu_v, pm_v, w_v,
             out_v, sem):
    wid = lax.axis_index("s") * _NC + lax.axis_index("c")
    base = wid * _BPW

    pltpu.sync_copy(w_hbm, w_v)
    pltpu.sync_copy(uix_hbm.at[pl.ds(base, _BPW)], idxu_v)
    pltpu.sync_copy(mix_hbm.at[pl.ds(base, _BPW)], idxm_v)

    for g in range(_BPW // _L):
        sl = pl.ds(g * _L, _L)
        gpu_v[sl] = lax.shift_right_logical(idxu_v[sl], 7)
        gpm_v[sl] = lax.shift_right_logical(idxm_v[sl], 7)

    copies = []
    for c in range(_NCH):
        sl = pl.ds(c * _CH, _CH)
        copies.append(pltpu.async_copy(
            umf_hbm.at[idxu_v.at[sl]], umf_v.at[sl], sem))
        copies.append(pltpu.async_copy(
            mmf_hbm.at[idxm_v.at[sl]], mmf_v.at[sl], sem))
    for cp in copies:
        cp.wait()

    wvecs = [w_v[pl.ds(j * _L, _L)] for j in range(2)]
    wmf = [wvecs[0][k] for k in range(_L)]
    c0v = wvecs[1]
    ii = lax.iota(jnp.int32, _L)
    kvecs = [jnp.full((_L,), k, jnp.int32) for k in range(_MF)]
    m127 = jnp.full((_L,), 127, jnp.int32)

    def chunk_body(c, carry):
        cb = c * _CH
        csl = pl.ds(cb, _CH)
        cpu = pltpu.async_copy(pu_hbm.at[gpu_v.at[csl]], pu_v, sem)
        cpm = pltpu.async_copy(pm_hbm.at[gpm_v.at[csl]], pm_v, sem)
        cpu.wait()
        cpm.wait()
        for g in range(_CH // _L):
            sl = pl.ds(cb + g * _L, _L)
            rows = cb + g * _L + ii
            lrows = g * _L + ii
            acc = c0v
            acc = acc + plsc.load_gather(
                pu_v, [lrows, jnp.bitwise_and(idxu_v[sl], m127)])
            acc = acc + plsc.load_gather(
                pm_v, [lrows, jnp.bitwise_and(idxm_v[sl], m127)])
            for k in range(_MF):
                u = plsc.load_gather(umf_v, [rows, kvecs[k]])
                m = plsc.load_gather(mmf_v, [rows, kvecs[k]])
                acc = acc + u * m * wmf[k]
            out_v[sl] = 1.0 / (1.0 + jnp.exp(-acc))
        return carry

    lax.fori_loop(0, _NCH, chunk_body, 0)
    pltpu.sync_copy(out_v, out_hbm.at[pl.ds(base, _BPW)])


_sc_call = functools.partial(
    pl.kernel,
    out_type=jax.ShapeDtypeStruct((_BATCH,), jnp.float32),
    mesh=plsc.VectorSubcoreMesh(core_axis_name="c", subcore_axis_name="s"),
    compiler_params=pltpu.CompilerParams(
        needs_layout_passes=False, use_tc_tiling_on_sc=False),
    scratch_types=[
        pltpu.VMEM((_BPW,), jnp.int32),
        pltpu.VMEM((_BPW,), jnp.int32),
        pltpu.VMEM((_BPW,), jnp.int32),
        pltpu.VMEM((_BPW,), jnp.int32),
        pltpu.VMEM((_BPW, _MF), jnp.float32),
        pltpu.VMEM((_BPW, _MF), jnp.float32),
        pltpu.VMEM((_CH, 128), jnp.float32),
        pltpu.VMEM((_CH, 128), jnp.float32),
        pltpu.VMEM((2 * _L,), jnp.float32),
        pltpu.VMEM((_BPW,), jnp.float32),
        pltpu.SemaphoreType.DMA,
    ],
)(_sc_body)


def kernel(X, user_mf, movie_mf, user_mlp, movie_mlp, W1, b1, W2, b2, Wf, bf):
    Xi = X.astype(jnp.int32)
    uix = Xi[:, 0]
    mix = Xi[:, 1]
    wf = Wf[0]
    wf_out = wf[_MF:]
    t = wf_out @ W2
    v = t @ W1
    c0 = jnp.dot(t, b1) + jnp.dot(wf_out, b2) + bf[0]
    wpack = jnp.concatenate([wf[:_MF], jnp.full((_L,), c0, jnp.float32)])
    pu = _matvec(user_mlp.T, v[:_MLP])
    pm = _matvec(movie_mlp.T, v[_MLP:])
    umf_pad = _hbm_pad8(user_mf)
    mmf_pad = _hbm_pad8(movie_mf)
    out = _sc_call(uix, mix, umf_pad, mmf_pad, pu, pm, wpack)
    return out.reshape(_BATCH, 1)


# speedup vs baseline: 37.9905x; 37.9905x over previous
import functools

import jax
import jax.numpy as jnp
from jax import lax
from jax.experimental import pallas as pl
from jax.experimental.pallas import tpu as pltpu
from jax.experimental.pallas import tpu_sc as plsc

_BATCH = 16384
_ROWS = 1000001
_MF = 16
_MLP = 32
_NC = 2
_NS = 16
_NW = _NC * _NS
_BPW = _BATCH // _NW
_CH = 128
_NCH = _BPW // _CH
_L = 16
_MVW = 16384
_NP = 62501
_MVOUT = _NP * _L


def _mv_body(t_ref, w_ref, out_ref):
    t = t_ref[...]
    w = w_ref[...]
    out_ref[...] = jnp.sum(t * w[:, None], axis=0)


def _matvec(table_t, w):
    flat = pl.pallas_call(
        _mv_body,
        grid=(pl.cdiv(_MVOUT, _MVW),),
        in_specs=[pl.BlockSpec((_MLP, _MVW), lambda j: (0, j)),
                  pl.BlockSpec((_MLP,), lambda j: (0,))],
        out_specs=pl.BlockSpec((_MVW,), lambda j: (j,)),
        out_shape=jax.ShapeDtypeStruct((_MVOUT,), jnp.float32),
    )(table_t, w)
    return flat.reshape(_NP, _L)


def _sc_body(uix_hbm, mix_hbm, umf_hbm, mmf_hbm, pu_hbm, pm_hbm, w_hbm,
             out_hbm,
             idxu_v, idxm_v, gpu_v, gpm_v, umf_v, mmf_v, pu_v, pm_v, w_v,
             out_v, sem):
    wid = lax.axis_index("s") * _NC + lax.axis_index("c")
    base = wid * _BPW

    pltpu.sync_copy(w_hbm, w_v)
    pltpu.sync_copy(uix_hbm.at[pl.ds(base, _BPW)], idxu_v)
    pltpu.sync_copy(mix_hbm.at[pl.ds(base, _BPW)], idxm_v)

    for g in range(_BPW // _L):
        sl = pl.ds(g * _L, _L)
        gpu_v[sl] = lax.shift_right_logical(idxu_v[sl], 4)
        gpm_v[sl] = lax.shift_right_logical(idxm_v[sl], 4)

    copies = []
    for c in range(_NCH):
        sl = pl.ds(c * _CH, _CH)
        copies.append(pltpu.async_copy(
            umf_hbm.at[idxu_v.at[sl]], umf_v.at[sl], sem))
        copies.append(pltpu.async_copy(
            mmf_hbm.at[idxm_v.at[sl]], mmf_v.at[sl], sem))
        copies.append(pltpu.async_copy(
            pu_hbm.at[gpu_v.at[sl]], pu_v.at[sl], sem))
        copies.append(pltpu.async_copy(
            pm_hbm.at[gpm_v.at[sl]], pm_v.at[sl], sem))
    for cp in copies:
        cp.wait()

    wvecs = [w_v[pl.ds(j * _L, _L)] for j in range(2)]
    wmf = [wvecs[0][k] for k in range(_L)]
    c0v = wvecs[1]
    ii = lax.iota(jnp.int32, _L)
    kvecs = [jnp.full((_L,), k, jnp.int32) for k in range(_MF)]
    m15 = jnp.full((_L,), 15, jnp.int32)

    def g_body(g, carry):
        sl = pl.ds(g * _L, _L)
        rows = g * _L + ii
        acc = c0v
        acc = acc + plsc.load_gather(
            pu_v, [rows, jnp.bitwise_and(idxu_v[sl], m15)])
        acc = acc + plsc.load_gather(
            pm_v, [rows, jnp.bitwise_and(idxm_v[sl], m15)])
        for k in range(_MF):
            u = plsc.load_gather(umf_v, [rows, kvecs[k]])
            m = plsc.load_gather(mmf_v, [rows, kvecs[k]])
            acc = acc + u * m * wmf[k]
        out_v[sl] = 1.0 / (1.0 + jnp.exp(-acc))
        return carry

    lax.fori_loop(0, _BPW // _L, g_body, 0)
    pltpu.sync_copy(out_v, out_hbm.at[pl.ds(base, _BPW)])


_sc_call = functools.partial(
    pl.kernel,
    out_type=jax.ShapeDtypeStruct((_BATCH,), jnp.float32),
    mesh=plsc.VectorSubcoreMesh(core_axis_name="c", subcore_axis_name="s"),
    compiler_params=pltpu.CompilerParams(
        needs_layout_passes=False, use_tc_tiling_on_sc=False),
    scratch_types=[
        pltpu.VMEM((_BPW,), jnp.int32),
        pltpu.VMEM((_BPW,), jnp.int32),
        pltpu.VMEM((_BPW,), jnp.int32),
        pltpu.VMEM((_BPW,), jnp.int32),
        pltpu.VMEM((_BPW, _MF), jnp.float32),
        pltpu.VMEM((_BPW, _MF), jnp.float32),
        pltpu.VMEM((_BPW, _L), jnp.float32),
        pltpu.VMEM((_BPW, _L), jnp.float32),
        pltpu.VMEM((2 * _L,), jnp.float32),
        pltpu.VMEM((_BPW,), jnp.float32),
        pltpu.SemaphoreType.DMA,
    ],
)(_sc_body)


def kernel(X, user_mf, movie_mf, user_mlp, movie_mlp, W1, b1, W2, b2, Wf, bf):
    Xi = X.astype(jnp.int32)
    uix = Xi[:, 0]
    mix = Xi[:, 1]
    wf = Wf[0]
    wf_out = wf[_MF:]
    t = wf_out @ W2
    v = t @ W1
    c0 = jnp.dot(t, b1) + jnp.dot(wf_out, b2) + bf[0]
    wpack = jnp.concatenate([wf[:_MF], jnp.full((_L,), c0, jnp.float32)])
    pu = _matvec(user_mlp.T, v[:_MLP])
    pm = _matvec(movie_mlp.T, v[_MLP:])
    out = _sc_call(uix, mix, user_mf, movie_mf, pu, pm, wpack)
    return out.reshape(_BATCH, 1)
